# Initial kernel scaffold; baseline (speedup 1.0000x reference)
import jax
import jax.numpy as jnp
from jax.experimental import pallas as pl


def kernel(x, edge_index, edge_attr, edge_types, W_src, W_dst, att_src, att_dst, W_edge, att_edge, edge_type_table, W_out, b_out, bias, ln_g, ln_b):
    raise NotImplementedError("write your pallas kernel here")


# speedup vs baseline: 42.8315x; 42.8315x over previous
import functools

import jax
import jax.numpy as jnp
import numpy as np
from jax import lax
from jax.experimental import pallas as pl
from jax.experimental.pallas import tpu as pltpu
from jax.experimental.pallas import tpu_sc as plsc

NC = 2
NS = 16
LANES = 16
NEG_SLOPE = 0.2


def _prep_body(x_ref, wst_ref, uu_ref, vv_ref, xs_ref, ga_ref, gb_ref):
    xb = x_ref[...]
    xs_ref[...] = jnp.dot(xb, wst_ref[...], preferred_element_type=jnp.float32)
    ga_ref[...] = jnp.dot(xb, uu_ref[...], preferred_element_type=jnp.float32)
    gb_ref[...] = jnp.dot(xb, vv_ref[...], preferred_element_type=jnp.float32)


def _run_prep(x, w_src_t, uu, vv, blk=1000):
    n, d = x.shape
    grid = n // blk
    return pl.pallas_call(
        _prep_body,
        grid=(grid,),
        in_specs=[
            pl.BlockSpec((blk, d), lambda i: (i, 0)),
            pl.BlockSpec((d, d), lambda i: (0, 0)),
            pl.BlockSpec((d, 16), lambda i: (0, 0)),
            pl.BlockSpec((d, 16), lambda i: (0, 0)),
        ],
        out_specs=[
            pl.BlockSpec((blk, d), lambda i: (i, 0)),
            pl.BlockSpec((blk, 16), lambda i: (i, 0)),
            pl.BlockSpec((blk, 16), lambda i: (i, 0)),
        ],
        out_shape=[
            jax.ShapeDtypeStruct((n, d), jnp.float32),
            jax.ShapeDtypeStruct((n, 16), jnp.float32),
            jax.ShapeDtypeStruct((n, 16), jnp.float32),
        ],
    )(x, w_src_t, uu, vv)


def _ebase_body(ea_ref, m_ref, o_ref):
    o_ref[...] = jnp.dot(ea_ref[...], m_ref[...],
                         preferred_element_type=jnp.float32)


def _run_ebase(ea, m, blk=10000):
    e, ed = ea.shape
    h = m.shape[1]
    grid = e // blk
    return pl.pallas_call(
        _ebase_body,
        grid=(grid,),
        in_specs=[
            pl.BlockSpec((blk, ed), lambda i: (i, 0)),
            pl.BlockSpec((ed, h), lambda i: (0, 0)),
        ],
        out_specs=pl.BlockSpec((blk, h), lambda i: (i, 0)),
        out_shape=jax.ShapeDtypeStruct((e, h), jnp.float32),
    )(ea, m)


def _make_sc_mp(n, e, d, h, nt, bsz):
    epw = e // (NC * NS)
    nb = epw // bsz
    rpt = (n // NS) // 8 * 8
    rlast = n - (NS - 1) * rpt
    ng = bsz // LANES
    mesh = plsc.VectorSubcoreMesh(core_axis_name="c", subcore_axis_name="s")

    def body(src_h, dst_h, ga_h, gb_h, xs_h, base_h, et_h, ttab_h, zacc_h,
             zes_h, acc_out, es_out,
             idx_s, idx_d, gd, gs, rows, baseb, etb, exb, ttab,
             accum, esum, sem1, sem2):
        c = lax.axis_index("c")
        s = lax.axis_index("s")
        w = c * NS + s

        pltpu.sync_copy(zacc_h.at[pl.ds(s * rpt, rpt)],
                        accum.at[pl.ds(s * rpt, rpt)])
        pltpu.sync_copy(zes_h.at[pl.ds(s * rpt, rpt)],
                        esum.at[pl.ds(s * rpt, rpt)])

        @pl.when(s == NS - 1)
        def _init_tail():
            off = (NS - 1) * rpt + rpt
            rem = rlast - rpt
            pltpu.sync_copy(zacc_h.at[pl.ds(off, rem)],
                            accum.at[pl.ds(off, rem)])
            pltpu.sync_copy(zes_h.at[pl.ds(off, rem)],
                            esum.at[pl.ds(off, rem)])

        pltpu.sync_copy(ttab_h, ttab)
        plsc.subcore_barrier()

        def batch_body(b, _):
            e0 = w * epw + b * bsz
            pltpu.sync_copy(src_h.at[pl.ds(e0, bsz)], idx_s)
            pltpu.sync_copy(dst_h.at[pl.ds(e0, bsz)], idx_d)
            pltpu.sync_copy(base_h.at[pl.ds(e0, bsz)], baseb)
            pltpu.sync_copy(et_h.at[pl.ds(e0, bsz)], etb)
            cp1 = pltpu.async_copy(ga_h.at[idx_d], gd, sem1)
            cp2 = pltpu.async_copy(gb_h.at[idx_s], gs, sem1)
            cp3 = pltpu.async_copy(xs_h.at[idx_s], rows, sem2)
            cp1.wait()
            cp2.wait()
            cp3.wait()

            def group_body(g, _):
                et_v = etb[pl.ds(g * LANES, LANES)]
                for l in range(LANES):
                    i = g * LANES + l
                    lg = (gd[i, :] + gs[i, :] + baseb[i, :]
                          + ttab[et_v[l], :])
                    lg = jnp.where(lg >= 0, lg, NEG_SLOPE * lg)
                    ex = jnp.exp(lg)
                    exb[i, :] = ex
                    for hh in range(h):
                        sc = jnp.broadcast_to(
                            lax.slice_in_dim(ex, hh, hh + 1), (LANES,))
                        rv = rows[i, pl.ds(hh * LANES, LANES)]
                        rows[i, pl.ds(hh * LANES, LANES)] = rv * sc
                return 0

            lax.fori_loop(0, ng, group_body, 0)

            pltpu.sync_copy(exb, esum.at[idx_d], add=True)
            pltpu.sync_copy(rows, accum.at[idx_d], add=True)
            return 0

        lax.fori_loop(0, nb, batch_body, 0)
        plsc.subcore_barrier()

        pltpu.sync_copy(accum.at[pl.ds(s * rpt, rpt)],
                        acc_out.at[c, pl.ds(s * rpt, rpt)])
        pltpu.sync_copy(esum.at[pl.ds(s * rpt, rpt)],
                        es_out.at[c, pl.ds(s * rpt, rpt)])

        @pl.when(s == NS - 1)
        def _dump_tail():
            off = (NS - 1) * rpt + rpt
            rem = rlast - rpt
            pltpu.sync_copy(accum.at[pl.ds(off, rem)],
                            acc_out.at[c, pl.ds(off, rem)])
            pltpu.sync_copy(esum.at[pl.ds(off, rem)],
                            es_out.at[c, pl.ds(off, rem)])

    return pl.kernel(
        body,
        out_type=(
            jax.ShapeDtypeStruct((NC, n, d), jnp.float32),
            jax.ShapeDtypeStruct((NC, n, 16), jnp.float32),
        ),
        mesh=mesh,
        scratch_types=[
            pltpu.VMEM((bsz,), jnp.int32),
            pltpu.VMEM((bsz,), jnp.int32),
            pltpu.VMEM((bsz, 16), jnp.float32),
            pltpu.VMEM((bsz, 16), jnp.float32),
            pltpu.VMEM((bsz, d), jnp.float32),
            pltpu.VMEM((bsz, 16), jnp.float32),
            pltpu.VMEM((bsz,), jnp.int32),
            pltpu.VMEM((bsz, 16), jnp.float32),
            pltpu.VMEM((nt, 16), jnp.float32),
            pltpu.VMEM_SHARED((n, d), jnp.float32),
            pltpu.VMEM_SHARED((n, 16), jnp.float32),
            pltpu.SemaphoreType.DMA,
            pltpu.SemaphoreType.DMA,
        ],
        compiler_params=pltpu.CompilerParams(use_tc_tiling_on_sc=False),
    )


def _final_body(a0_ref, a1_ref, e0_ref, e1_ref, ga_ref, gb_ref, xs_ref,
                x_ref, cl_ref, wot_ref, bb_ref, lng_ref, lnb_ref, r8_ref,
                o_ref):
    h = 8
    ai = ga_ref[:, :h]
    aj = gb_ref[:, :h]
    lg = ai + aj + cl_ref[0:1, :h]
    lg = jnp.where(lg >= 0, lg, NEG_SLOPE * lg)
    exl = jnp.exp(lg)
    es = e0_ref[:, :h] + e1_ref[:, :h] + exl
    r8 = r8_ref[...]
    acc = (a0_ref[...] + a1_ref[...]
           + jnp.dot(exl, r8, preferred_element_type=jnp.float32)
           * xs_ref[...])
    recip = 1.0 / (es + 1e-16)
    outp = acc * jnp.dot(recip, r8, preferred_element_type=jnp.float32)
    y = jnp.dot(outp, wot_ref[...],
                preferred_element_type=jnp.float32) + bb_ref[...]
    mu = jnp.mean(y, axis=-1, keepdims=True)
    yc = y - mu
    var = jnp.mean(yc * yc, axis=-1, keepdims=True)
    y = yc * lax.rsqrt(var + 1e-5) * lng_ref[...] + lnb_ref[...]
    o_ref[...] = y + x_ref[...]


def _run_final(a0, a1, e0, e1, ga, gb, xs, x, cl, wot, bb, lng, lnb, r8,
               blk=1000):
    n, d = x.shape
    grid = n // blk
    row = lambda i: (i, 0)
    full = lambda i: (0, 0)
    return pl.pallas_call(
        _final_body,
        grid=(grid,),
        in_specs=[
            pl.BlockSpec((blk, d), row),
            pl.BlockSpec((blk, d), row),
            pl.BlockSpec((blk, 16), row),
            pl.BlockSpec((blk, 16), row),
            pl.BlockSpec((blk, 16), row),
            pl.BlockSpec((blk, 16), row),
            pl.BlockSpec((blk, d), row),
            pl.BlockSpec((blk, d), row),
            pl.BlockSpec((1, 16), full),
            pl.BlockSpec((d, d), full),
            pl.BlockSpec((1, d), full),
            pl.BlockSpec((1, d), full),
            pl.BlockSpec((1, d), full),
            pl.BlockSpec((8, d), full),
        ],
        out_specs=pl.BlockSpec((blk, d), row),
        out_shape=jax.ShapeDtypeStruct((n, d), jnp.float32),
    )(a0, a1, e0, e1, ga, gb, xs, x, cl, wot, bb, lng, lnb, r8)


def kernel(x, edge_index, edge_attr, edge_types, W_src, W_dst, att_src,
           att_dst, W_edge, att_edge, edge_type_table, W_out, b_out, bias,
           ln_g, ln_b):
    n, d = x.shape
    e = edge_index.shape[1]
    h, c = att_src.shape[1], att_src.shape[2]
    ed = edge_attr.shape[1]
    nt = edge_type_table.shape[0]

    f32 = jnp.float32
    wd3 = W_dst.reshape(h, c, d)
    ws3 = W_src.reshape(h, c, d)
    we3 = W_edge.reshape(h, c, ed)
    u = jnp.einsum("hcd,hc->dh", wd3, att_src[0])
    v = jnp.einsum("hcd,hc->dh", ws3, att_dst[0])
    uu = jnp.concatenate([u, u], axis=1)
    vv = jnp.concatenate([v, v], axis=1)
    m = jnp.einsum("hck,hc->kh", we3, att_edge[0])
    m16 = jnp.concatenate([m, m], axis=1)
    tt3 = edge_type_table.reshape(nt, h, c)
    ttab = jnp.einsum("thc,hc->th", tt3, att_edge[0])
    ttab16 = jnp.concatenate([ttab, ttab], axis=1)
    cl = jnp.sum(m, axis=0) + ttab[nt - 1]
    cl2 = jnp.zeros((1, 16), f32).at[0, :h].set(cl)

    src = edge_index[0].astype(jnp.int32)
    dst = edge_index[1].astype(jnp.int32)

    xs, ga, gb = _run_prep(x, W_src.T, uu, vv)
    base = _run_ebase(edge_attr, m16)

    sc_mp = _make_sc_mp(n, e, d, h, nt, bsz=80)
    acc2, es2 = sc_mp(src, dst, ga, gb, xs, base,
                      edge_types.astype(jnp.int32), ttab16,
                      jnp.zeros((n, d), f32), jnp.zeros((n, 16), f32))

    r8 = jnp.asarray(np.kron(np.eye(h), np.ones((1, c))), f32)
    out = _run_final(acc2[0], acc2[1], es2[0], es2[1], ga, gb, xs, x,
                     cl2, W_out.T, (b_out + bias)[None, :],
                     ln_g[None, :], ln_b[None, :], r8)
    return out


# speedup vs baseline: 46.0273x; 1.0746x over previous
import functools

import jax
import jax.numpy as jnp
import numpy as np
from jax import lax
from jax.experimental import pallas as pl
from jax.experimental.pallas import tpu as pltpu
from jax.experimental.pallas import tpu_sc as plsc

NC = 2
NS = 16
LANES = 16
NEG_SLOPE = 0.2


def _prep_body(x_ref, wst_ref, uu_ref, vv_ref, xs_ref, ga_ref, gb_ref):
    xb = x_ref[...]
    xs_ref[...] = jnp.dot(xb, wst_ref[...], preferred_element_type=jnp.float32)
    ga_ref[...] = jnp.dot(xb, uu_ref[...], preferred_element_type=jnp.float32)
    gb_ref[...] = jnp.dot(xb, vv_ref[...], preferred_element_type=jnp.float32)


def _run_prep(x, w_src_t, uu, vv, blk=1000):
    n, d = x.shape
    grid = n // blk
    return pl.pallas_call(
        _prep_body,
        grid=(grid,),
        in_specs=[
            pl.BlockSpec((blk, d), lambda i: (i, 0)),
            pl.BlockSpec((d, d), lambda i: (0, 0)),
            pl.BlockSpec((d, 16), lambda i: (0, 0)),
            pl.BlockSpec((d, 16), lambda i: (0, 0)),
        ],
        out_specs=[
            pl.BlockSpec((blk, d), lambda i: (i, 0)),
            pl.BlockSpec((blk, 16), lambda i: (i, 0)),
            pl.BlockSpec((blk, 16), lambda i: (i, 0)),
        ],
        out_shape=[
            jax.ShapeDtypeStruct((n, d), jnp.float32),
            jax.ShapeDtypeStruct((n, 16), jnp.float32),
            jax.ShapeDtypeStruct((n, 16), jnp.float32),
        ],
    )(x, w_src_t, uu, vv)


def _ebase_body(ea_ref, et_ref, m_ref, t_ref, o_ref):
    etv = et_ref[...]
    tio = lax.broadcasted_iota(jnp.int32, (1, 8), 1)
    oh = (etv == tio).astype(jnp.float32)
    o_ref[...] = (jnp.dot(ea_ref[...], m_ref[...],
                          preferred_element_type=jnp.float32)
                  + jnp.dot(oh, t_ref[...],
                            preferred_element_type=jnp.float32))


def _run_ebase(ea, et2, m, t8, blk=10000):
    e, ed = ea.shape
    w = m.shape[1]
    grid = e // blk
    return pl.pallas_call(
        _ebase_body,
        grid=(grid,),
        in_specs=[
            pl.BlockSpec((blk, ed), lambda i: (i, 0)),
            pl.BlockSpec((blk, 1), lambda i: (i, 0)),
            pl.BlockSpec((ed, w), lambda i: (0, 0)),
            pl.BlockSpec((8, w), lambda i: (0, 0)),
        ],
        out_specs=pl.BlockSpec((blk, w), lambda i: (i, 0)),
        out_shape=jax.ShapeDtypeStruct((e, w), jnp.float32),
    )(ea, et2, m, t8)


def _make_sc_mp(n, e, d, h, nt, bsz):
    epw = e // (NC * NS)
    nb = epw // bsz
    rpt = (n // NS) // 8 * 8
    rlast = n - (NS - 1) * rpt
    ng = bsz // LANES
    mesh = plsc.VectorSubcoreMesh(core_axis_name="c", subcore_axis_name="s")

    def body(src_h, dst_h, ga_h, gb_h, xs_h, base_h, zacc_h,
             zes_h, acc_out, es_out,
             idx_s, idx_d, gd, gs, rows, baseb, exb,
             accum, esum, sem1, sem2):
        c = lax.axis_index("c")
        s = lax.axis_index("s")
        w = c * NS + s

        pltpu.sync_copy(zacc_h.at[pl.ds(s * rpt, rpt)],
                        accum.at[pl.ds(s * rpt, rpt)])
        pltpu.sync_copy(zes_h.at[pl.ds(s * rpt, rpt)],
                        esum.at[pl.ds(s * rpt, rpt)])

        @pl.when(s == NS - 1)
        def _init_tail():
            off = (NS - 1) * rpt + rpt
            rem = rlast - rpt
            pltpu.sync_copy(zacc_h.at[pl.ds(off, rem)],
                            accum.at[pl.ds(off, rem)])
            pltpu.sync_copy(zes_h.at[pl.ds(off, rem)],
                            esum.at[pl.ds(off, rem)])

        plsc.subcore_barrier()

        def batch_body(b, _):
            e0 = w * epw + b * bsz
            pltpu.sync_copy(src_h.at[pl.ds(e0, bsz)], idx_s)
            pltpu.sync_copy(dst_h.at[pl.ds(e0, bsz)], idx_d)
            pltpu.sync_copy(base_h.at[pl.ds(e0, bsz)], baseb)
            cp1 = pltpu.async_copy(ga_h.at[idx_d], gd, sem1)
            cp2 = pltpu.async_copy(gb_h.at[idx_s], gs, sem1)
            cp3 = pltpu.async_copy(xs_h.at[idx_s], rows, sem2)
            cp1.wait()
            cp2.wait()
            cp3.wait()

            for i in range(bsz):
                lg = gd[i, :] + gs[i, :] + baseb[i, :]
                lg = jnp.where(lg >= 0, lg, NEG_SLOPE * lg)
                ex = jnp.exp(lg)
                exb[i, :] = ex
                for hh in range(h):
                    sc = jnp.broadcast_to(
                        lax.slice_in_dim(ex, hh, hh + 1), (LANES,))
                    rv = rows[i, pl.ds(hh * LANES, LANES)]
                    rows[i, pl.ds(hh * LANES, LANES)] = rv * sc

            pltpu.sync_copy(exb, esum.at[idx_d], add=True)
            pltpu.sync_copy(rows, accum.at[idx_d], add=True)
            return 0

        lax.fori_loop(0, nb, batch_body, 0)
        plsc.subcore_barrier()

        pltpu.sync_copy(accum.at[pl.ds(s * rpt, rpt)],
                        acc_out.at[c, pl.ds(s * rpt, rpt)])
        pltpu.sync_copy(esum.at[pl.ds(s * rpt, rpt)],
                        es_out.at[c, pl.ds(s * rpt, rpt)])

        @pl.when(s == NS - 1)
        def _dump_tail():
            off = (NS - 1) * rpt + rpt
            rem = rlast - rpt
            pltpu.sync_copy(accum.at[pl.ds(off, rem)],
                            acc_out.at[c, pl.ds(off, rem)])
            pltpu.sync_copy(esum.at[pl.ds(off, rem)],
                            es_out.at[c, pl.ds(off, rem)])

    return pl.kernel(
        body,
        out_type=(
            jax.ShapeDtypeStruct((NC, n, d), jnp.float32),
            jax.ShapeDtypeStruct((NC, n, 16), jnp.float32),
        ),
        mesh=mesh,
        scratch_types=[
            pltpu.VMEM((bsz,), jnp.int32),
            pltpu.VMEM((bsz,), jnp.int32),
            pltpu.VMEM((bsz, 16), jnp.float32),
            pltpu.VMEM((bsz, 16), jnp.float32),
            pltpu.VMEM((bsz, d), jnp.float32),
            pltpu.VMEM((bsz, 16), jnp.float32),
            pltpu.VMEM((bsz, 16), jnp.float32),
            pltpu.VMEM_SHARED((n, d), jnp.float32),
            pltpu.VMEM_SHARED((n, 16), jnp.float32),
            pltpu.SemaphoreType.DMA,
            pltpu.SemaphoreType.DMA,
        ],
        compiler_params=pltpu.CompilerParams(use_tc_tiling_on_sc=False),
    )


def _final_body(a0_ref, a1_ref, e0_ref, e1_ref, ga_ref, gb_ref, xs_ref,
                x_ref, cl_ref, wot_ref, bb_ref, lng_ref, lnb_ref, r8_ref,
                o_ref):
    h = 8
    ai = ga_ref[:, :h]
    aj = gb_ref[:, :h]
    lg = ai + aj + cl_ref[0:1, :h]
    lg = jnp.where(lg >= 0, lg, NEG_SLOPE * lg)
    exl = jnp.exp(lg)
    es = e0_ref[:, :h] + e1_ref[:, :h] + exl
    r8 = r8_ref[...]
    acc = (a0_ref[...] + a1_ref[...]
           + jnp.dot(exl, r8, preferred_element_type=jnp.float32)
           * xs_ref[...])
    recip = 1.0 / (es + 1e-16)
    outp = acc * jnp.dot(recip, r8, preferred_element_type=jnp.float32)
    y = jnp.dot(outp, wot_ref[...],
                preferred_element_type=jnp.float32) + bb_ref[...]
    mu = jnp.mean(y, axis=-1, keepdims=True)
    yc = y - mu
    var = jnp.mean(yc * yc, axis=-1, keepdims=True)
    y = yc * lax.rsqrt(var + 1e-5) * lng_ref[...] + lnb_ref[...]
    o_ref[...] = y + x_ref[...]


def _run_final(a0, a1, e0, e1, ga, gb, xs, x, cl, wot, bb, lng, lnb, r8,
               blk=1000):
    n, d = x.shape
    grid = n // blk
    row = lambda i: (i, 0)
    full = lambda i: (0, 0)
    return pl.pallas_call(
        _final_body,
        grid=(grid,),
        in_specs=[
            pl.BlockSpec((blk, d), row),
            pl.BlockSpec((blk, d), row),
            pl.BlockSpec((blk, 16), row),
            pl.BlockSpec((blk, 16), row),
            pl.BlockSpec((blk, 16), row),
            pl.BlockSpec((blk, 16), row),
            pl.BlockSpec((blk, d), row),
            pl.BlockSpec((blk, d), row),
            pl.BlockSpec((1, 16), full),
            pl.BlockSpec((d, d), full),
            pl.BlockSpec((1, d), full),
            pl.BlockSpec((1, d), full),
            pl.BlockSpec((1, d), full),
            pl.BlockSpec((8, d), full),
        ],
        out_specs=pl.BlockSpec((blk, d), row),
        out_shape=jax.ShapeDtypeStruct((n, d), jnp.float32),
    )(a0, a1, e0, e1, ga, gb, xs, x, cl, wot, bb, lng, lnb, r8)


def kernel(x, edge_index, edge_attr, edge_types, W_src, W_dst, att_src,
           att_dst, W_edge, att_edge, edge_type_table, W_out, b_out, bias,
           ln_g, ln_b):
    n, d = x.shape
    e = edge_index.shape[1]
    h, c = att_src.shape[1], att_src.shape[2]
    ed = edge_attr.shape[1]
    nt = edge_type_table.shape[0]

    f32 = jnp.float32
    wd3 = W_dst.reshape(h, c, d)
    ws3 = W_src.reshape(h, c, d)
    we3 = W_edge.reshape(h, c, ed)
    u = jnp.einsum("hcd,hc->dh", wd3, att_src[0])
    v = jnp.einsum("hcd,hc->dh", ws3, att_dst[0])
    uu = jnp.concatenate([u, u], axis=1)
    vv = jnp.concatenate([v, v], axis=1)
    m = jnp.einsum("hck,hc->kh", we3, att_edge[0])
    m16 = jnp.concatenate([m, m], axis=1)
    tt3 = edge_type_table.reshape(nt, h, c)
    ttab = jnp.einsum("thc,hc->th", tt3, att_edge[0])
    ttab16 = jnp.concatenate([ttab, ttab], axis=1)
    ttab8 = jnp.zeros((8, 16), f32).at[:nt].set(ttab16)
    cl = jnp.sum(m, axis=0) + ttab[nt - 1]
    cl2 = jnp.zeros((1, 16), f32).at[0, :h].set(cl)

    src = edge_index[0].astype(jnp.int32)
    dst = edge_index[1].astype(jnp.int32)

    xs, ga, gb = _run_prep(x, W_src.T, uu, vv)
    base = _run_ebase(edge_attr, edge_types.astype(jnp.int32)[:, None],
                      m16, ttab8)

    sc_mp = _make_sc_mp(n, e, d, h, nt, bsz=80)
    acc2, es2 = sc_mp(src, dst, ga, gb, xs, base,
                      jnp.zeros((n, d), f32), jnp.zeros((n, 16), f32))

    r8 = jnp.asarray(np.kron(np.eye(h), np.ones((1, c))), f32)
    out = _run_final(acc2[0], acc2[1], es2[0], es2[1], ga, gb, xs, x,
                     cl2, W_out.T, (b_out + bias)[None, :],
                     ln_g[None, :], ln_b[None, :], r8)
    return out
